# Initial kernel scaffold; baseline (speedup 1.0000x reference)
import jax
import jax.numpy as jnp
from jax.experimental import pallas as pl


def kernel(x, edge_index, W1, b1, W2, b2, W3, b3):
    raise NotImplementedError("write your pallas kernel here")


# speedup vs baseline: 54.4806x; 54.4806x over previous
import functools

import jax
import jax.numpy as jnp
from jax import lax
from jax.experimental import pallas as pl
from jax.experimental.pallas import tpu as pltpu
from jax.experimental.pallas import tpu_sc as plsc

N_NODES = 100000
PAD_IDX = N_NODES
NC, NS = 2, 16
NW = NC * NS
NCOL = 100352
CPT = NCOL // NS
EDGE_COLS = 128
E_PAD_ROWS = 50176
ROWS_PER_WORKER = E_PAD_ROWS // NW
E_PAD = E_PAD_ROWS * EDGE_COLS
K = 8

_mesh = functools.partial(
    plsc.VectorSubcoreMesh,
    core_axis_name="c", subcore_axis_name="s", num_cores=NC, num_subcores=NS)


def _sc_pass(cols, src1d, dst2d, zeros, *, F, stage_table, ones_gather=False):
    nblocks = ROWS_PER_WORKER // K
    BLK = K * EDGE_COLS
    n_tab = 0 if ones_gather else F
    scratch = (
        ([] if ones_gather else [pltpu.VMEM((BLK,), jnp.int32)])
        + [pltpu.VMEM((K, EDGE_COLS), jnp.int32)]
        + ([pltpu.VMEM((EDGE_COLS,), jnp.float32)] if ones_gather
           else [pltpu.VMEM((BLK,), jnp.float32) for _ in range(F)])
        + [pltpu.VMEM_SHARED((NCOL,), jnp.float32) for _ in range(F)]
        + ([pltpu.VMEM_SHARED((NCOL,), jnp.float32) for _ in range(n_tab)]
           if stage_table else [])
        + [pltpu.SemaphoreType.DMA] * 3
    )

    @functools.partial(
        pl.kernel,
        out_type=jax.ShapeDtypeStruct((NC * F * NCOL,), jnp.float32),
        mesh=_mesh(),
        scratch_types=scratch,
    )
    def k(*refs):
        tabs_hbm = refs[0:n_tab]
        src_hbm = None if ones_gather else refs[n_tab]
        pos = n_tab + (0 if ones_gather else 1)
        dst_hbm, zero_hbm = refs[pos], refs[pos + 1]
        out = refs[pos + 2]
        base = pos + 3
        if ones_gather:
            didx = refs[base]
            base += 1
        else:
            sidx, didx = refs[base], refs[base + 1]
            base += 2
        msgs = refs[base: base + F]
        base += F
        accs = refs[base: base + F]
        base += F
        if stage_table and not ones_gather:
            tabs = refs[base: base + F]
            base += F
        else:
            tabs = tabs_hbm
        isem, gsem, ssem = refs[base], refs[base + 1], refs[base + 2]

        cid = lax.axis_index("c")
        sid = lax.axis_index("s")
        wid = cid * NS + sid
        sl = pl.ds(sid * CPT, CPT)
        for f in range(F):
            pltpu.sync_copy(zero_hbm.at[sl], accs[f].at[sl])
            if stage_table and not ones_gather:
                pltpu.sync_copy(tabs_hbm[f].at[sl], tabs[f].at[sl])
        if ones_gather:
            for i in range(EDGE_COLS // 16):
                msgs[0][pl.ds(i * 16, 16)] = jnp.full((16,), 1.0, jnp.float32)
        plsc.subcore_barrier()
        row0 = wid * ROWS_PER_WORKER

        @pl.loop(0, nblocks)
        def _(b):
            hd = pltpu.async_copy(
                dst_hbm.at[pl.ds(row0 + b * K, K)], didx, isem)
            if not ones_gather:
                hs = pltpu.async_copy(
                    src_hbm.at[pl.ds((row0 + b * K) * EDGE_COLS, BLK)],
                    sidx, isem)
                hs.wait()
                hd.wait()
                gh = [pltpu.async_copy(tabs[f].at[sidx], msgs[f], gsem)
                      for f in range(F)]
                for h in gh:
                    h.wait()
            else:
                hd.wait()

            if ones_gather:
                sh = [pltpu.async_copy(msgs[0], accs[0].at[didx.at[j]],
                                       ssem, add=True)
                      for j in range(K)]
            else:
                sh = [pltpu.async_copy(
                    msgs[f].at[pl.ds(j * EDGE_COLS, EDGE_COLS)],
                    accs[f].at[didx.at[j]], ssem, add=True)
                    for j in range(K) for f in range(F)]
            for h in sh:
                h.wait()

        plsc.subcore_barrier()
        for f in range(F):
            pltpu.sync_copy(
                accs[f].at[sl],
                out.at[pl.ds((cid * F + f) * NCOL + sid * CPT, CPT)])

    args = ([] if ones_gather else list(cols)) + (
        [dst2d, zeros] if ones_gather else [src1d, dst2d, zeros])
    return k(*args).reshape(NC * F, NCOL)


_CB = 12544
_G = NCOL // _CB


def _stage_a(degp, x_c):
    def body(dp_ref, x_ref, dinv_ref, xp_ref):
        deg = dp_ref[0:1, :] + dp_ref[1:2, :] + 1.0
        dinv = lax.rsqrt(deg)
        dinv_ref[...] = dinv
        xp_ref[...] = x_ref[...] * dinv

    return pl.pallas_call(
        body,
        grid=(_G,),
        in_specs=[pl.BlockSpec((2, _CB), lambda i: (0, i)),
                  pl.BlockSpec((2, _CB), lambda i: (0, i))],
        out_specs=[pl.BlockSpec((1, _CB), lambda i: (0, i)),
                   pl.BlockSpec((2, _CB), lambda i: (0, i))],
        out_shape=[jax.ShapeDtypeStruct((1, NCOL), jnp.float32),
                   jax.ShapeDtypeStruct((2, NCOL), jnp.float32)],
    )(degp, x_c)


def _stage_b(s1p, xp_c, dinv_c, W1t, b1c):
    def body(p_ref, xp_ref, dinv_ref, w_ref, b_ref, out_ref):
        dinv = dinv_ref[...]
        pre = (p_ref[0:2, :] + p_ref[2:4, :] + xp_ref[...]) * dinv
        w = w_ref[...]
        h = (w[:, 0:1] * pre[0:1, :] + w[:, 1:2] * pre[1:2, :] + b_ref[...])
        out_ref[...] = jnp.maximum(h, 0.0) * dinv

    return pl.pallas_call(
        body,
        grid=(_G,),
        in_specs=[pl.BlockSpec((4, _CB), lambda i: (0, i)),
                  pl.BlockSpec((2, _CB), lambda i: (0, i)),
                  pl.BlockSpec((1, _CB), lambda i: (0, i)),
                  pl.BlockSpec((16, 2), lambda i: (0, 0)),
                  pl.BlockSpec((16, 1), lambda i: (0, 0))],
        out_specs=pl.BlockSpec((16, _CB), lambda i: (0, i)),
        out_shape=jax.ShapeDtypeStruct((16, NCOL), jnp.float32),
    )(s1p, xp_c, dinv_c, W1t, b1c)


def _stage_c(s2p, h1p_c, dinv_c, W2t, b2c, W3t):
    def body(p_ref, h_ref, dinv_ref, w2_ref, b2_ref, w3_ref, out_ref):
        dinv = dinv_ref[...]
        pre = (p_ref[0:16, :] + p_ref[16:32, :] + h_ref[...]) * dinv
        h2 = jnp.dot(w2_ref[...], pre, preferred_element_type=jnp.float32)
        h2 = jnp.maximum(h2 + b2_ref[...], 0.0)
        g = jnp.dot(w3_ref[...], h2, preferred_element_type=jnp.float32)
        out_ref[...] = g * dinv

    return pl.pallas_call(
        body,
        grid=(_G,),
        in_specs=[pl.BlockSpec((32, _CB), lambda i: (0, i)),
                  pl.BlockSpec((16, _CB), lambda i: (0, i)),
                  pl.BlockSpec((1, _CB), lambda i: (0, i)),
                  pl.BlockSpec((32, 16), lambda i: (0, 0)),
                  pl.BlockSpec((32, 1), lambda i: (0, 0)),
                  pl.BlockSpec((1, 32), lambda i: (0, 0))],
        out_specs=pl.BlockSpec((1, _CB), lambda i: (0, i)),
        out_shape=jax.ShapeDtypeStruct((1, NCOL), jnp.float32),
    )(s2p, h1p_c, dinv_c, W2t, b2c, W3t)


def _stage_d(s3p, gp_c, dinv_c, b3):
    def body(p_ref, g_ref, dinv_ref, b_ref, out_ref):
        out_ref[...] = ((p_ref[0:1, :] + p_ref[1:2, :] + g_ref[...])
                        * dinv_ref[...] + b_ref[...])

    return pl.pallas_call(
        body,
        grid=(_G,),
        in_specs=[pl.BlockSpec((2, _CB), lambda i: (0, i)),
                  pl.BlockSpec((1, _CB), lambda i: (0, i)),
                  pl.BlockSpec((1, _CB), lambda i: (0, i)),
                  pl.BlockSpec((1, 1), lambda i: (0, 0))],
        out_specs=pl.BlockSpec((1, _CB), lambda i: (0, i)),
        out_shape=jax.ShapeDtypeStruct((1, NCOL), jnp.float32),
    )(s3p, gp_c, dinv_c, b3)


def _cols(mat_c):
    return [mat_c[f] for f in range(mat_c.shape[0])]


def kernel(x, edge_index, W1, b1, W2, b2, W3, b3):
    src = edge_index[0].astype(jnp.int32)
    dst = edge_index[1].astype(jnp.int32)
    pad_e = E_PAD - src.shape[0]
    src1d = jnp.pad(src, (0, pad_e), constant_values=PAD_IDX)
    dst2d = jnp.pad(dst, (0, pad_e), constant_values=PAD_IDX).reshape(-1, EDGE_COLS)
    x_c = jnp.pad(x.T, ((0, 0), (0, NCOL - N_NODES)))
    zeros = jnp.zeros((NCOL,), jnp.float32)

    degp = _sc_pass(None, None, dst2d, zeros,
                    F=1, stage_table=False, ones_gather=True)
    dinv_c, xp_c = _stage_a(degp, x_c)

    s1 = _sc_pass(_cols(xp_c), src1d, dst2d, zeros, F=2, stage_table=True)
    h1p_c = _stage_b(s1, xp_c, dinv_c, W1.T, b1.reshape(16, 1))

    h1_cols = _cols(h1p_c)
    s2a = _sc_pass(h1_cols[0:8], src1d, dst2d, zeros, F=8, stage_table=True)
    s2b = _sc_pass(h1_cols[8:16], src1d, dst2d, zeros, F=8, stage_table=True)
    s2 = jnp.concatenate([s2a[0:8], s2b[0:8], s2a[8:16], s2b[8:16]])
    gp_c = _stage_c(s2, h1p_c, dinv_c, W2.T, b2.reshape(32, 1), W3.T)

    s3 = _sc_pass([gp_c[0]], src1d, dst2d, zeros, F=1, stage_table=True)
    out_c = _stage_d(s3, gp_c, dinv_c, b3.reshape(1, 1))
    return out_c[0, :N_NODES].reshape(N_NODES, 1)
